# Initial kernel scaffold; baseline (speedup 1.0000x reference)
import jax
import jax.numpy as jnp
from jax.experimental import pallas as pl


def kernel(input_ids, token_type_ids, age, abspos, concept_table, segment_table, age_w0, age_b0, age_w, age_b, abspos_w0, abspos_b0, abspos_w, abspos_b, ln_gamma, ln_beta):
    raise NotImplementedError("write your pallas kernel here")


# speedup vs baseline: 2.3855x; 2.3855x over previous
import functools

import jax
import jax.numpy as jnp
from jax import lax
from jax.experimental import pallas as pl
from jax.experimental.pallas import tpu as pltpu
from jax.experimental.pallas import tpu_sc as plsc

B, S, V, T, H = 1024, 200, 100000, 2, 128
N = B * S
EPS = 1e-12

NW = 32
ROWS_PER_W = N // NW
IDXROWS = N // 128
IDXROWS_PER_W = IDXROWS // NW
CHUNK = 256
NCHUNK = ROWS_PER_W // CHUNK


def _sc_gather(ids2d, table):
    mesh = plsc.VectorSubcoreMesh(core_axis_name="c", subcore_axis_name="s")

    @functools.partial(
        pl.kernel,
        mesh=mesh,
        out_type=jax.ShapeDtypeStruct((N, H), jnp.float32),
        scratch_types=[
            pltpu.VMEM((IDXROWS_PER_W, 128), jnp.int32),
            pltpu.VMEM((CHUNK, H), jnp.float32),
            pltpu.SemaphoreType.DMA,
        ],
    )
    def k(ids_hbm, table_hbm, out_hbm, idx_v, rows_v, sem):
        wid = lax.axis_index("s") * 2 + lax.axis_index("c")
        pltpu.sync_copy(ids_hbm.at[wid], idx_v)

        def body(c, carry):
            cp0 = pltpu.async_copy(
                table_hbm.at[idx_v.at[2 * c]], rows_v.at[pl.ds(0, 128)], sem)
            cp1 = pltpu.async_copy(
                table_hbm.at[idx_v.at[2 * c + 1]], rows_v.at[pl.ds(128, 128)],
                sem)
            cp0.wait()
            cp1.wait()
            pltpu.sync_copy(
                rows_v, out_hbm.at[pl.ds(wid * ROWS_PER_W + c * CHUNK, CHUNK)])
            return carry

        lax.fori_loop(0, NCHUNK, body, 0)

    return k(ids2d, table)


def _tc_fuse(gathered, age, abspos, tt, params):
    TB = 512

    def body(g_ref, age_ref, ab_ref, tt_ref, p_ref, out_ref):
        p = p_ref[:]
        wa, ba, wb, bb = p[0:1], p[1:2], p[2:3], p[3:4]
        gamma, beta, seg0, seg1 = p[4:5], p[5:6], p[6:7], p[7:8]
        rows = g_ref[:]
        age_v = age_ref[:]
        ab_v = ab_ref[:]
        tt_v = tt_ref[:]
        va = age_v * wa + ba
        vb = ab_v * wb + bb
        col = lax.broadcasted_iota(jnp.int32, (1, H), 1)
        t2v_a = jnp.where(col == 0, va, jnp.sin(va))
        t2v_b = jnp.where(col == 0, vb, jnp.sin(vb))
        emb = rows + jnp.where(tt_v == 0, seg0, seg1) + t2v_a + t2v_b
        mean = jnp.mean(emb, axis=1, keepdims=True)
        cen = emb - mean
        var = jnp.mean(cen * cen, axis=1, keepdims=True)
        out_ref[:] = cen * lax.rsqrt(var + EPS) * gamma + beta

    return pl.pallas_call(
        body,
        grid=(N // TB,),
        in_specs=[
            pl.BlockSpec((TB, H), lambda i: (i, 0)),
            pl.BlockSpec((TB, 1), lambda i: (i, 0)),
            pl.BlockSpec((TB, 1), lambda i: (i, 0)),
            pl.BlockSpec((TB, 1), lambda i: (i, 0)),
            pl.BlockSpec((8, H), lambda i: (0, 0)),
        ],
        out_specs=pl.BlockSpec((TB, H), lambda i: (i, 0)),
        out_shape=jax.ShapeDtypeStruct((N, H), jnp.float32),
    )(gathered, age, abspos, tt, params)


def kernel(input_ids, token_type_ids, age, abspos, concept_table,
           segment_table, age_w0, age_b0, age_w, age_b,
           abspos_w0, abspos_b0, abspos_w, abspos_b, ln_gamma, ln_beta):
    ids2d = input_ids.reshape(NW, IDXROWS_PER_W, 128).astype(jnp.int32)
    gathered = _sc_gather(ids2d, concept_table)
    params = jnp.stack([
        jnp.concatenate([age_w0, age_w]),
        jnp.concatenate([age_b0, age_b]),
        jnp.concatenate([abspos_w0, abspos_w]),
        jnp.concatenate([abspos_b0, abspos_b]),
        ln_gamma, ln_beta,
        segment_table[0], segment_table[1],
    ])
    out = _tc_fuse(
        gathered,
        age.reshape(N, 1),
        abspos.reshape(N, 1),
        token_type_ids.reshape(N, 1).astype(jnp.int32),
        params,
    )
    return out.reshape(B, S, H)


# speedup vs baseline: 3.8477x; 1.6130x over previous
import functools

import jax
import jax.numpy as jnp
from jax import lax
from jax.experimental import pallas as pl
from jax.experimental.pallas import tpu as pltpu
from jax.experimental.pallas import tpu_sc as plsc

B, S, V, T, H = 1024, 200, 100000, 2, 128
N = B * S
EPS = 1e-12

NW = 32
ROWS_PER_W = N // NW
IDXROWS = N // 128
IDXROWS_PER_W = IDXROWS // NW
CHUNK = 256
NCHUNK = ROWS_PER_W // CHUNK


def _sc_gather(ids2d, table):
    mesh = plsc.VectorSubcoreMesh(core_axis_name="c", subcore_axis_name="s")

    @functools.partial(
        pl.kernel,
        mesh=mesh,
        out_type=jax.ShapeDtypeStruct((N, H), jnp.float32),
        scratch_types=[
            pltpu.VMEM((IDXROWS_PER_W, 128), jnp.int32),
            pltpu.VMEM((CHUNK, H), jnp.float32),
            pltpu.SemaphoreType.DMA,
        ],
    )
    def k(ids_hbm, table_hbm, out_hbm, idx_v, rows_v, sem):
        wid = lax.axis_index("s") * 2 + lax.axis_index("c")
        pltpu.sync_copy(ids_hbm.at[wid], idx_v)

        def body(c, carry):
            cp0 = pltpu.async_copy(
                table_hbm.at[idx_v.at[2 * c]], rows_v.at[pl.ds(0, 128)], sem)
            cp1 = pltpu.async_copy(
                table_hbm.at[idx_v.at[2 * c + 1]], rows_v.at[pl.ds(128, 128)],
                sem)
            cp0.wait()
            cp1.wait()
            pltpu.sync_copy(
                rows_v, out_hbm.at[pl.ds(wid * ROWS_PER_W + c * CHUNK, CHUNK)])
            return carry

        lax.fori_loop(0, NCHUNK, body, 0)

    return k(ids2d, table)


def _fast_sin(x):
    n = x * 0.15915494309189535
    half = jnp.where(n >= 0, jnp.float32(0.5), jnp.float32(-0.5))
    k = (n + half).astype(jnp.int32).astype(jnp.float32)
    r = (x - k * 6.28125) - k * 0.0019353071795864769
    r2 = r * r
    p = jnp.float32(-2.0176527e-08)
    p = p * r2 + 2.6948044e-06
    p = p * r2 + -0.0001980393
    p = p * r2 + 0.008332207
    p = p * r2 + -0.1666652
    p = p * r2 + 0.99999946
    return p * r


def _tc_fuse(gathered, age, abspos, tt, params):
    TB = 512

    def body(g_ref, age_ref, ab_ref, tt_ref, p_ref, out_ref):
        p = p_ref[:]
        wa, ba, wb, bb = p[0:1], p[1:2], p[2:3], p[3:4]
        gamma, beta, seg0, seg1 = p[4:5], p[5:6], p[6:7], p[7:8]
        rows = g_ref[:]
        age_v = age_ref[:]
        ab_v = ab_ref[:]
        tt_v = tt_ref[:]
        va = age_v * wa + ba
        vb = ab_v * wb + bb
        col = lax.broadcasted_iota(jnp.int32, (1, H), 1)
        t2v_a = jnp.where(col == 0, va, _fast_sin(va))
        t2v_b = jnp.where(col == 0, vb, _fast_sin(vb))
        emb = rows + jnp.where(tt_v == 0, seg0, seg1) + t2v_a + t2v_b
        mean = jnp.mean(emb, axis=1, keepdims=True)
        cen = emb - mean
        var = jnp.mean(cen * cen, axis=1, keepdims=True)
        out_ref[:] = cen * lax.rsqrt(var + EPS) * gamma + beta

    return pl.pallas_call(
        body,
        grid=(N // TB,),
        in_specs=[
            pl.BlockSpec((TB, H), lambda i: (i, 0)),
            pl.BlockSpec((TB, 1), lambda i: (i, 0)),
            pl.BlockSpec((TB, 1), lambda i: (i, 0)),
            pl.BlockSpec((TB, 1), lambda i: (i, 0)),
            pl.BlockSpec((8, H), lambda i: (0, 0)),
        ],
        out_specs=pl.BlockSpec((TB, H), lambda i: (i, 0)),
        out_shape=jax.ShapeDtypeStruct((N, H), jnp.float32),
    )(gathered, age, abspos, tt, params)


def kernel(input_ids, token_type_ids, age, abspos, concept_table,
           segment_table, age_w0, age_b0, age_w, age_b,
           abspos_w0, abspos_b0, abspos_w, abspos_b, ln_gamma, ln_beta):
    ids2d = input_ids.reshape(NW, IDXROWS_PER_W, 128).astype(jnp.int32)
    gathered = _sc_gather(ids2d, concept_table)
    params = jnp.stack([
        jnp.concatenate([age_w0, age_w]),
        jnp.concatenate([age_b0, age_b]),
        jnp.concatenate([abspos_w0, abspos_w]),
        jnp.concatenate([abspos_b0, abspos_b]),
        ln_gamma, ln_beta,
        segment_table[0], segment_table[1],
    ])
    out = _tc_fuse(
        gathered,
        age.reshape(N, 1),
        abspos.reshape(N, 1),
        token_type_ids.reshape(N, 1).astype(jnp.int32),
        params,
    )
    return out.reshape(B, S, H)


# speedup vs baseline: 6.2699x; 1.6295x over previous
import functools

import jax
import jax.numpy as jnp
from jax import lax
from jax.experimental import pallas as pl
from jax.experimental.pallas import tpu as pltpu
from jax.experimental.pallas import tpu_sc as plsc

B, S, V, T, H = 1024, 200, 100000, 2, 128
N = B * S
EPS = 1e-12

NW = 32
ROWS_PER_W = N // NW
IDXROWS = N // 128
IDXROWS_PER_W = IDXROWS // NW
CHUNK = 256
NCHUNK = ROWS_PER_W // CHUNK


def _sc_gather(ids2d, table):
    mesh = plsc.VectorSubcoreMesh(core_axis_name="c", subcore_axis_name="s")

    @functools.partial(
        pl.kernel,
        mesh=mesh,
        out_type=jax.ShapeDtypeStruct((N, H), jnp.float32),
        scratch_types=[
            pltpu.VMEM((IDXROWS_PER_W, 128), jnp.int32),
            pltpu.VMEM((CHUNK, H), jnp.float32),
            pltpu.SemaphoreType.DMA,
        ],
    )
    def k(ids_hbm, table_hbm, out_hbm, idx_v, rows_v, sem):
        wid = lax.axis_index("s") * 2 + lax.axis_index("c")
        pltpu.sync_copy(ids_hbm.at[wid], idx_v)

        def body(c, carry):
            cp0 = pltpu.async_copy(
                table_hbm.at[idx_v.at[2 * c]], rows_v.at[pl.ds(0, 128)], sem)
            cp1 = pltpu.async_copy(
                table_hbm.at[idx_v.at[2 * c + 1]], rows_v.at[pl.ds(128, 128)],
                sem)
            cp0.wait()
            cp1.wait()
            pltpu.sync_copy(
                rows_v, out_hbm.at[pl.ds(wid * ROWS_PER_W + c * CHUNK, CHUNK)])
            return carry

        lax.fori_loop(0, NCHUNK, body, 0)

    return k(ids2d, table)


def _fast_sin(x):
    k = jnp.floor(x * 0.15915494309189535 + 0.5)
    r = x - k * 6.2831855
    r2 = r * r
    p = jnp.float32(2.1183632e-06)
    p = p * r2 + -0.00019201539
    p = p * r2 + 0.008304462
    p = p * r2 + -0.16661254
    p = p * r2 + 0.9999711
    return p * r


def _tc_fuse(gathered, age, abspos, tt, params):
    BR = 8
    TB = BR * S

    def _outer(col_v, row_v):
        return lax.dot_general(col_v, row_v, (((1,), (0,)), ((), ())),
                               precision=lax.Precision.DEFAULT)

    def body(g_ref, age_ref, ab_ref, tt_ref, p_ref, out_ref):
        p = p_ref[:]
        wa, ba, wb, bb = p[0:1], p[1:2], p[2:3], p[3:4]
        gamma, beta, seg0, seg1 = p[4:5], p[5:6], p[6:7], p[7:8]
        dseg = seg1 - seg0
        age_t = jnp.transpose(age_ref[:])
        ab_t = jnp.transpose(ab_ref[:])
        tt_t = jnp.transpose(tt_ref[:]).astype(jnp.float32)
        col = lax.broadcasted_iota(jnp.int32, (1, H), 1)
        mones = jnp.full((H, H), 1.0 / H, jnp.float32)
        for j in range(BR):
            sl = pl.ds(j * S, S)
            rows = g_ref[sl, :]
            va = age_t[:, j:j + 1] * wa + ba
            vb = ab_t[:, j:j + 1] * wb + bb
            seg = _outer(tt_t[:, j:j + 1], dseg) + seg0
            t2v = jnp.where(col == 0, va + vb,
                            _fast_sin(va) + _fast_sin(vb))
            emb = rows + seg + t2v
            mean_bc = lax.dot_general(emb, mones, (((1,), (0,)), ((), ())),
                                      precision=lax.Precision.DEFAULT)
            cen = emb - mean_bc
            var_bc = lax.dot_general(cen * cen, mones,
                                     (((1,), (0,)), ((), ())),
                                     precision=lax.Precision.DEFAULT)
            out_ref[sl, :] = cen * lax.rsqrt(var_bc + EPS) * gamma + beta

    return pl.pallas_call(
        body,
        grid=(B // BR,),
        in_specs=[
            pl.BlockSpec((TB, H), lambda i: (i, 0)),
            pl.BlockSpec((BR, S), lambda i: (i, 0)),
            pl.BlockSpec((BR, S), lambda i: (i, 0)),
            pl.BlockSpec((BR, S), lambda i: (i, 0)),
            pl.BlockSpec((8, H), lambda i: (0, 0)),
        ],
        out_specs=pl.BlockSpec((TB, H), lambda i: (i, 0)),
        out_shape=jax.ShapeDtypeStruct((N, H), jnp.float32),
    )(gathered, age, abspos, tt, params)


def kernel(input_ids, token_type_ids, age, abspos, concept_table,
           segment_table, age_w0, age_b0, age_w, age_b,
           abspos_w0, abspos_b0, abspos_w, abspos_b, ln_gamma, ln_beta):
    ids2d = input_ids.reshape(NW, IDXROWS_PER_W, 128).astype(jnp.int32)
    gathered = _sc_gather(ids2d, concept_table)
    params = jnp.stack([
        jnp.concatenate([age_w0, age_w]),
        jnp.concatenate([age_b0, age_b]),
        jnp.concatenate([abspos_w0, abspos_w]),
        jnp.concatenate([abspos_b0, abspos_b]),
        ln_gamma, ln_beta,
        segment_table[0], segment_table[1],
    ])
    out = _tc_fuse(gathered, age, abspos,
                   token_type_ids.astype(jnp.int32), params)
    return out.reshape(B, S, H)


# speedup vs baseline: 7.0984x; 1.1321x over previous
import functools

import jax
import jax.numpy as jnp
from jax import lax
from jax.experimental import pallas as pl
from jax.experimental.pallas import tpu as pltpu
from jax.experimental.pallas import tpu_sc as plsc

B, S, V, T, H = 1024, 200, 100000, 2, 128
N = B * S
EPS = 1e-12

NW = 32
ROWS_PER_W = N // NW
IDXROWS = N // 128
IDXROWS_PER_W = IDXROWS // NW
CHUNK = 256
NCHUNK = ROWS_PER_W // CHUNK


def _sc_gather(ids2d, table):
    mesh = plsc.VectorSubcoreMesh(core_axis_name="c", subcore_axis_name="s")

    @functools.partial(
        pl.kernel,
        mesh=mesh,
        out_type=jax.ShapeDtypeStruct((N, H), jnp.float32),
        scratch_types=[
            pltpu.VMEM((IDXROWS_PER_W, 128), jnp.int32),
            pltpu.VMEM((CHUNK, H), jnp.float32),
            pltpu.SemaphoreType.DMA,
        ],
    )
    def k(ids_hbm, table_hbm, out_hbm, idx_v, rows_v, sem):
        wid = lax.axis_index("s") * 2 + lax.axis_index("c")
        pltpu.sync_copy(ids_hbm.at[wid], idx_v)

        def body(c, carry):
            cp0 = pltpu.async_copy(
                table_hbm.at[idx_v.at[2 * c]], rows_v.at[pl.ds(0, 128)], sem)
            cp1 = pltpu.async_copy(
                table_hbm.at[idx_v.at[2 * c + 1]], rows_v.at[pl.ds(128, 128)],
                sem)
            cp0.wait()
            cp1.wait()
            pltpu.sync_copy(
                rows_v, out_hbm.at[pl.ds(wid * ROWS_PER_W + c * CHUNK, CHUNK)])
            return carry

        lax.fori_loop(0, NCHUNK, body, 0)

    return k(ids2d, table)


def _fast_sin(x):
    k = jnp.floor(x * 0.15915494309189535 + 0.5)
    r = x - k * 6.2831855
    r2 = r * r
    p = jnp.float32(2.1183632e-06)
    p = p * r2 + -0.00019201539
    p = p * r2 + 0.008304462
    p = p * r2 + -0.16661254
    p = p * r2 + 0.9999711
    return p * r


def _tc_fuse(gathered, age, abspos, tt, params):
    BR = 32
    TB = BR * S

    def _outer(col_v, row_v):
        return lax.dot_general(col_v, row_v, (((1,), (0,)), ((), ())),
                               precision=lax.Precision.DEFAULT)

    def body(g_ref, age_ref, ab_ref, tt_ref, p_ref, out_ref):
        p = p_ref[:]
        wa, ba, wb, bb = p[0:1], p[1:2], p[2:3], p[3:4]
        gamma, beta, seg0, seg1 = p[4:5], p[5:6], p[6:7], p[7:8]
        dseg = seg1 - seg0
        age_t = jnp.transpose(age_ref[:])
        ab_t = jnp.transpose(ab_ref[:])
        tt_t = jnp.transpose(tt_ref[:]).astype(jnp.float32)
        col = lax.broadcasted_iota(jnp.int32, (1, H), 1)
        mones = jnp.full((H, H), 1.0 / H, jnp.float32)
        for j in range(BR):
            sl = pl.ds(j * S, S)
            rows = g_ref[sl, :]
            va = age_t[:, j:j + 1] * wa + ba
            vb = ab_t[:, j:j + 1] * wb + bb
            seg = _outer(tt_t[:, j:j + 1], dseg) + seg0
            t2v = jnp.where(col == 0, va + vb,
                            _fast_sin(va) + _fast_sin(vb))
            emb = rows + seg + t2v
            mean_bc = lax.dot_general(emb, mones, (((1,), (0,)), ((), ())),
                                      precision=lax.Precision.DEFAULT)
            cen = emb - mean_bc
            var_bc = lax.dot_general(cen * cen, mones,
                                     (((1,), (0,)), ((), ())),
                                     precision=lax.Precision.DEFAULT)
            out_ref[sl, :] = cen * lax.rsqrt(var_bc + EPS) * gamma + beta

    return pl.pallas_call(
        body,
        grid=(B // BR,),
        in_specs=[
            pl.BlockSpec((TB, H), lambda i: (i, 0)),
            pl.BlockSpec((BR, S), lambda i: (i, 0)),
            pl.BlockSpec((BR, S), lambda i: (i, 0)),
            pl.BlockSpec((BR, S), lambda i: (i, 0)),
            pl.BlockSpec((8, H), lambda i: (0, 0)),
        ],
        out_specs=pl.BlockSpec((TB, H), lambda i: (i, 0)),
        out_shape=jax.ShapeDtypeStruct((N, H), jnp.float32),
    )(gathered, age, abspos, tt, params)


def kernel(input_ids, token_type_ids, age, abspos, concept_table,
           segment_table, age_w0, age_b0, age_w, age_b,
           abspos_w0, abspos_b0, abspos_w, abspos_b, ln_gamma, ln_beta):
    ids2d = input_ids.reshape(NW, IDXROWS_PER_W, 128).astype(jnp.int32)
    gathered = _sc_gather(ids2d, concept_table)
    params = jnp.stack([
        jnp.concatenate([age_w0, age_w]),
        jnp.concatenate([age_b0, age_b]),
        jnp.concatenate([abspos_w0, abspos_w]),
        jnp.concatenate([abspos_b0, abspos_b]),
        ln_gamma, ln_beta,
        segment_table[0], segment_table[1],
    ])
    out = _tc_fuse(gathered, age, abspos,
                   token_type_ids.astype(jnp.int32), params)
    return out.reshape(B, S, H)


# speedup vs baseline: 8.2063x; 1.1561x over previous
import functools

import jax
import jax.numpy as jnp
from jax import lax
from jax.experimental import pallas as pl
from jax.experimental.pallas import tpu as pltpu
from jax.experimental.pallas import tpu_sc as plsc

B, S, V, T, H = 1024, 200, 100000, 2, 128
N = B * S
EPS = 1e-12

NW = 32
ROWS_PER_W = N // NW
IDXROWS = N // 128
IDXROWS_PER_W = IDXROWS // NW
CHUNK = 256
NCHUNK = ROWS_PER_W // CHUNK


def _sc_gather(ids3d, table):
    _, R, _ = ids3d.shape
    G = 5
    CH = G * 128
    rows_per_w = R * 128
    mesh = plsc.VectorSubcoreMesh(core_axis_name="c", subcore_axis_name="s")

    @functools.partial(
        pl.kernel,
        mesh=mesh,
        out_type=jax.ShapeDtypeStruct((NW * rows_per_w, H), jnp.float32),
        scratch_types=[
            pltpu.VMEM((R, 128), jnp.int32),
            pltpu.VMEM((CH, H), jnp.float32),
            pltpu.SemaphoreType.DMA,
        ],
    )
    def k(ids_hbm, table_hbm, out_hbm, idx_v, rows_v, sem):
        wid = lax.axis_index("s") * 2 + lax.axis_index("c")
        pltpu.sync_copy(ids_hbm.at[wid], idx_v)

        def body(c, carry):
            cps = [
                pltpu.async_copy(
                    table_hbm.at[idx_v.at[G * c + g]],
                    rows_v.at[pl.ds(g * 128, 128)], sem)
                for g in range(G)
            ]
            for cp in cps:
                cp.wait()
            pltpu.sync_copy(
                rows_v, out_hbm.at[pl.ds(wid * rows_per_w + c * CH, CH)])
            return carry

        lax.fori_loop(0, R // G, body, 0)

    return k(ids3d, table)


def _fast_sin(x):
    k = jnp.floor(x * 0.15915494309189535 + 0.5)
    r = x - k * 6.2831855
    r2 = r * r
    p = jnp.float32(2.1183632e-06)
    p = p * r2 + -0.00019201539
    p = p * r2 + 0.008304462
    p = p * r2 + -0.16661254
    p = p * r2 + 0.9999711
    return p * r


def _tc_fuse(gathered, age, abspos, tt, params, piece, npieces, out_prev):
    BR = 32
    TB = BR * S

    def _outer(col_v, row_v):
        return lax.dot_general(col_v, row_v, (((1,), (0,)), ((), ())),
                               precision=lax.Precision.DEFAULT)

    def body(g_ref, age_ref, ab_ref, tt_ref, p_ref, out_ref):
        p = p_ref[:]
        wa, ba, wb, bb = p[0:1], p[1:2], p[2:3], p[3:4]
        gamma, beta, seg0, seg1 = p[4:5], p[5:6], p[6:7], p[7:8]
        dseg = seg1 - seg0
        age_t = jnp.transpose(age_ref[:])
        ab_t = jnp.transpose(ab_ref[:])
        tt_t = jnp.transpose(tt_ref[:]).astype(jnp.float32)
        col = lax.broadcasted_iota(jnp.int32, (1, H), 1)
        mones = jnp.full((H, H), 1.0 / H, jnp.float32)
        for j in range(BR):
            sl = pl.ds(j * S, S)
            rows = g_ref[sl, :]
            va = age_t[:, j:j + 1] * wa + ba
            vb = ab_t[:, j:j + 1] * wb + bb
            seg = _outer(tt_t[:, j:j + 1], dseg) + seg0
            t2v = jnp.where(col == 0, va + vb,
                            _fast_sin(va) + _fast_sin(vb))
            emb = rows + seg + t2v
            mean_bc = lax.dot_general(emb, mones, (((1,), (0,)), ((), ())),
                                      precision=lax.Precision.DEFAULT)
            cen = emb - mean_bc
            var_bc = lax.dot_general(cen * cen, mones,
                                     (((1,), (0,)), ((), ())),
                                     precision=lax.Precision.DEFAULT)
            out_ref[sl, :] = cen * lax.rsqrt(var_bc + EPS) * gamma + beta

    nblk = B // npieces // BR
    off = piece * nblk
    in_specs = [
        pl.BlockSpec((TB, H), lambda i: (i, 0)),
        pl.BlockSpec((BR, S), lambda i: (i + off, 0)),
        pl.BlockSpec((BR, S), lambda i: (i + off, 0)),
        pl.BlockSpec((BR, S), lambda i: (i + off, 0)),
        pl.BlockSpec((8, H), lambda i: (0, 0)),
    ]
    args = [gathered, age, abspos, tt, params]
    aliases = {}
    if out_prev is not None:
        in_specs.append(pl.BlockSpec(memory_space=pl.ANY))
        args.append(out_prev)
        aliases = {5: 0}

    def wrapped(*refs):
        body(*refs[:5], refs[-1])

    return pl.pallas_call(
        wrapped,
        grid=(nblk,),
        in_specs=in_specs,
        out_specs=pl.BlockSpec((TB, H), lambda i: (i + off, 0)),
        out_shape=jax.ShapeDtypeStruct((N, H), jnp.float32),
        input_output_aliases=aliases,
    )(*args)


def kernel(input_ids, token_type_ids, age, abspos, concept_table,
           segment_table, age_w0, age_b0, age_w, age_b,
           abspos_w0, abspos_b0, abspos_w, abspos_b, ln_gamma, ln_beta):
    P = 2
    BP = B // P
    params = jnp.stack([
        jnp.concatenate([age_w0, age_w]),
        jnp.concatenate([age_b0, age_b]),
        jnp.concatenate([abspos_w0, abspos_w]),
        jnp.concatenate([abspos_b0, abspos_b]),
        ln_gamma, ln_beta,
        segment_table[0], segment_table[1],
    ])
    ids = input_ids.astype(jnp.int32)
    tt = token_type_ids.astype(jnp.int32)
    gathered = [
        _sc_gather(
            ids[p * BP:(p + 1) * BP].reshape(NW, BP * S // NW // 128, 128),
            concept_table)
        for p in range(P)
    ]
    out = None
    for p in range(P):
        out = _tc_fuse(gathered[p], age, abspos, tt, params, p, P, out)
    return out.reshape(B, S, H)


# speedup vs baseline: 10.1116x; 1.2322x over previous
import functools

import jax
import jax.numpy as jnp
from jax import lax
from jax.experimental import pallas as pl
from jax.experimental.pallas import tpu as pltpu
from jax.experimental.pallas import tpu_sc as plsc

B, S, V, T, H = 1024, 200, 100000, 2, 128
N = B * S
EPS = 1e-12

NW = 32
ROWS_PER_W = N // NW
IDXROWS = N // 128
IDXROWS_PER_W = IDXROWS // NW
CHUNK = 256
NCHUNK = ROWS_PER_W // CHUNK


def _sc_gather(ids3d, table):
    _, R, _ = ids3d.shape
    G = 5
    CH = G * 128
    rows_per_w = R * 128
    mesh = plsc.VectorSubcoreMesh(core_axis_name="c", subcore_axis_name="s")

    @functools.partial(
        pl.kernel,
        mesh=mesh,
        out_type=jax.ShapeDtypeStruct((NW * rows_per_w, H), jnp.float32),
        scratch_types=[
            pltpu.VMEM((R, 128), jnp.int32),
            pltpu.VMEM((CH, H), jnp.float32),
            pltpu.SemaphoreType.DMA,
        ],
    )
    def k(ids_hbm, table_hbm, out_hbm, idx_v, rows_v, sem):
        wid = lax.axis_index("s") * 2 + lax.axis_index("c")
        pltpu.sync_copy(ids_hbm.at[wid], idx_v)

        def body(c, carry):
            cps = [
                pltpu.async_copy(
                    table_hbm.at[idx_v.at[G * c + g]],
                    rows_v.at[pl.ds(g * 128, 128)], sem)
                for g in range(G)
            ]
            for cp in cps:
                cp.wait()
            pltpu.sync_copy(
                rows_v, out_hbm.at[pl.ds(wid * rows_per_w + c * CH, CH)])
            return carry

        lax.fori_loop(0, R // G, body, 0)

    return k(ids3d, table)


def _fast_sin(x):
    k = jnp.floor(x * 0.15915494309189535 + 0.5)
    r = x - k * 6.2831855
    r2 = r * r
    p = jnp.float32(2.1183632e-06)
    p = p * r2 + -0.00019201539
    p = p * r2 + 0.008304462
    p = p * r2 + -0.16661254
    p = p * r2 + 0.9999711
    return p * r


def _tc_fuse(gathered, age, abspos, tt, params, piece, npieces, out_prev):
    BR = 32
    TB = BR * S

    def _outer(col_v, row_v):
        return lax.dot_general(col_v, row_v, (((1,), (0,)), ((), ())),
                               precision=lax.Precision.DEFAULT)

    def body(g_ref, age_ref, ab_ref, tt_ref, p_ref, out_ref):
        p = p_ref[:]
        wa, ba, wb, bb = p[0:1], p[1:2], p[2:3], p[3:4]
        gamma, beta, seg0, seg1 = p[4:5], p[5:6], p[6:7], p[7:8]
        dseg = seg1 - seg0
        age_t = jnp.transpose(age_ref[:])
        ab_t = jnp.transpose(ab_ref[:])
        tt_t = jnp.transpose(tt_ref[:]).astype(jnp.float32)
        col = lax.broadcasted_iota(jnp.int32, (1, H), 1)
        mones = jnp.full((H, H), 1.0 / H, jnp.float32)
        for j in range(BR):
            sl = pl.ds(j * S, S)
            rows = g_ref[sl, :]
            va = age_t[:, j:j + 1] * wa + ba
            vb = ab_t[:, j:j + 1] * wb + bb
            seg = jnp.where(tt_t[:, j:j + 1] == 0, seg0, seg1)
            t2v = jnp.where(col == 0, va + vb,
                            _fast_sin(va) + _fast_sin(vb))
            emb = rows + seg + t2v
            mean_bc = lax.dot_general(emb, mones, (((1,), (0,)), ((), ())),
                                      precision=lax.Precision.DEFAULT)
            cen = emb - mean_bc
            var_bc = lax.dot_general(cen * cen, mones,
                                     (((1,), (0,)), ((), ())),
                                     precision=lax.Precision.DEFAULT)
            out_ref[sl, :] = cen * lax.rsqrt(var_bc + EPS) * gamma + beta

    nblk = B // npieces // BR
    off = piece * nblk
    in_specs = [
        pl.BlockSpec((TB, H), lambda i: (i, 0)),
        pl.BlockSpec((BR, S), lambda i: (i + off, 0)),
        pl.BlockSpec((BR, S), lambda i: (i + off, 0)),
        pl.BlockSpec((BR, S), lambda i: (i + off, 0)),
        pl.BlockSpec((8, H), lambda i: (0, 0)),
    ]
    args = [gathered, age, abspos, tt, params]
    aliases = {}
    if out_prev is not None:
        in_specs.append(pl.BlockSpec(memory_space=pl.ANY))
        args.append(out_prev)
        aliases = {5: 0}

    def wrapped(*refs):
        body(*refs[:5], refs[-1])

    return pl.pallas_call(
        wrapped,
        grid=(nblk,),
        in_specs=in_specs,
        out_specs=pl.BlockSpec((TB, H), lambda i: (i + off, 0)),
        out_shape=jax.ShapeDtypeStruct((N, H), jnp.float32),
        input_output_aliases=aliases,
    )(*args)


def kernel(input_ids, token_type_ids, age, abspos, concept_table,
           segment_table, age_w0, age_b0, age_w, age_b,
           abspos_w0, abspos_b0, abspos_w, abspos_b, ln_gamma, ln_beta):
    P = 2
    BP = B // P
    params = jnp.stack([
        jnp.concatenate([age_w0, age_w]),
        jnp.concatenate([age_b0, age_b]),
        jnp.concatenate([abspos_w0, abspos_w]),
        jnp.concatenate([abspos_b0, abspos_b]),
        ln_gamma, ln_beta,
        segment_table[0], segment_table[1],
    ])
    ids = input_ids.astype(jnp.int32)
    tt = token_type_ids.astype(jnp.int32)
    gathered = [
        _sc_gather(
            ids[p * BP:(p + 1) * BP].reshape(NW, BP * S // NW // 128, 128),
            concept_table)
        for p in range(P)
    ]
    out = None
    for p in range(P):
        out = _tc_fuse(gathered[p], age, abspos, tt, params, p, P, out)
    return out.reshape(B, S, H)


# speedup vs baseline: 10.6688x; 1.0551x over previous
import functools

import jax
import jax.numpy as jnp
from jax import lax
from jax.experimental import pallas as pl
from jax.experimental.pallas import tpu as pltpu
from jax.experimental.pallas import tpu_sc as plsc

B, S, V, T, H = 1024, 200, 100000, 2, 128
N = B * S
EPS = 1e-12

NW = 32
ROWS_PER_W = N // NW
IDXROWS = N // 128
IDXROWS_PER_W = IDXROWS // NW
CHUNK = 256
NCHUNK = ROWS_PER_W // CHUNK


def _sc_gather(ids3d, table):
    _, R, L = ids3d.shape
    G = 5
    CH = G * L
    rows_per_w = R * L
    mesh = plsc.VectorSubcoreMesh(core_axis_name="c", subcore_axis_name="s")

    @functools.partial(
        pl.kernel,
        mesh=mesh,
        out_type=jax.ShapeDtypeStruct((NW * rows_per_w, H), jnp.float32),
        scratch_types=[
            pltpu.VMEM((R, L), jnp.int32),
            pltpu.VMEM((CH, H), jnp.float32),
            pltpu.SemaphoreType.DMA,
        ],
    )
    def k(ids_hbm, table_hbm, out_hbm, idx_v, rows_v, sem):
        wid = lax.axis_index("s") * 2 + lax.axis_index("c")
        pltpu.sync_copy(ids_hbm.at[wid], idx_v)

        def body(c, carry):
            cps = [
                pltpu.async_copy(
                    table_hbm.at[idx_v.at[G * c + g]],
                    rows_v.at[pl.ds(g * L, L)], sem)
                for g in range(G)
            ]
            for cp in cps:
                cp.wait()
            pltpu.sync_copy(
                rows_v, out_hbm.at[pl.ds(wid * rows_per_w + c * CH, CH)])
            return carry

        lax.fori_loop(0, R // G, body, 0)

    return k(ids3d, table)


def _fast_sin(x):
    k = jnp.floor(x * 0.15915494309189535 + 0.5)
    r = x - k * 6.2831855
    r2 = r * r
    p = jnp.float32(-0.00014183763)
    p = p * r2 + 0.007904465
    p = p * r2 + -0.16541623
    p = p * r2 + 0.9989872
    return p * r


def _tc_fuse(gathered, age, abspos, tt, params, piece, npieces, out_prev):
    BR = 32
    TB = BR * S

    def _outer(col_v, row_v):
        return lax.dot_general(col_v, row_v, (((1,), (0,)), ((), ())),
                               precision=lax.Precision.DEFAULT)

    def body(g_ref, age_ref, ab_ref, tt_ref, p_ref, out_ref):
        p = p_ref[:]
        wa, ba, wb, bb = p[0:1], p[1:2], p[2:3], p[3:4]
        gamma, beta, seg0, seg1 = p[4:5], p[5:6], p[6:7], p[7:8]
        dseg = seg1 - seg0
        age_t = jnp.transpose(age_ref[:])
        ab_t = jnp.transpose(ab_ref[:])
        tt_t = jnp.transpose(tt_ref[:]).astype(jnp.float32)
        col = lax.broadcasted_iota(jnp.int32, (1, H), 1)
        mones = jnp.full((H, H), 1.0 / H, jnp.float32)
        for j in range(BR):
            sl = pl.ds(j * S, S)
            rows = g_ref[sl, :]
            va = age_t[:, j:j + 1] * wa + ba
            vb = ab_t[:, j:j + 1] * wb + bb
            seg = jnp.where(tt_t[:, j:j + 1] == 0, seg0, seg1)
            t2v = jnp.where(col == 0, va + vb,
                            _fast_sin(va) + _fast_sin(vb))
            emb = rows + seg + t2v
            mean_bc = lax.dot_general(emb, mones, (((1,), (0,)), ((), ())),
                                      precision=lax.Precision.DEFAULT)
            cen = emb - mean_bc
            var_bc = lax.dot_general(cen * cen, mones,
                                     (((1,), (0,)), ((), ())),
                                     precision=lax.Precision.DEFAULT)
            out_ref[sl, :] = cen * lax.rsqrt(var_bc + EPS) * gamma + beta

    nblk = B // npieces // BR
    off = piece * nblk
    in_specs = [
        pl.BlockSpec((TB, H), lambda i: (i, 0)),
        pl.BlockSpec((BR, S), lambda i: (i + off, 0)),
        pl.BlockSpec((BR, S), lambda i: (i + off, 0)),
        pl.BlockSpec((BR, S), lambda i: (i + off, 0)),
        pl.BlockSpec((8, H), lambda i: (0, 0)),
    ]
    args = [gathered, age, abspos, tt, params]
    aliases = {}
    if out_prev is not None:
        in_specs.append(pl.BlockSpec(memory_space=pl.ANY))
        args.append(out_prev)
        aliases = {5: 0}

    def wrapped(*refs):
        body(*refs[:5], refs[-1])

    return pl.pallas_call(
        wrapped,
        grid=(nblk,),
        in_specs=in_specs,
        out_specs=pl.BlockSpec((TB, H), lambda i: (i + off, 0)),
        out_shape=jax.ShapeDtypeStruct((N, H), jnp.float32),
        input_output_aliases=aliases,
    )(*args)


def kernel(input_ids, token_type_ids, age, abspos, concept_table,
           segment_table, age_w0, age_b0, age_w, age_b,
           abspos_w0, abspos_b0, abspos_w, abspos_b, ln_gamma, ln_beta):
    P = 4
    BP = B // P
    L = BP * S // NW // 25
    params = jnp.stack([
        jnp.concatenate([age_w0, age_w]),
        jnp.concatenate([age_b0, age_b]),
        jnp.concatenate([abspos_w0, abspos_w]),
        jnp.concatenate([abspos_b0, abspos_b]),
        ln_gamma, ln_beta,
        segment_table[0], segment_table[1],
    ])
    ids = input_ids.astype(jnp.int32)
    tt = token_type_ids.astype(jnp.int32)
    gathered = [
        _sc_gather(
            ids[p * BP:(p + 1) * BP].reshape(NW, 25, L),
            concept_table)
        for p in range(P)
    ]
    out = None
    for p in range(P):
        out = _tc_fuse(gathered[p], age, abspos, tt, params, p, P, out)
    return out.reshape(B, S, H)


# speedup vs baseline: 11.0575x; 1.0364x over previous
import functools

import jax
import jax.numpy as jnp
from jax import lax
from jax.experimental import pallas as pl
from jax.experimental.pallas import tpu as pltpu
from jax.experimental.pallas import tpu_sc as plsc

B, S, V, T, H = 1024, 200, 100000, 2, 128
N = B * S
EPS = 1e-12

NW = 32
ROWS_PER_W = N // NW
IDXROWS = N // 128
IDXROWS_PER_W = IDXROWS // NW
CHUNK = 256
NCHUNK = ROWS_PER_W // CHUNK


def _sc_gather(ids3d, table):
    _, R, L = ids3d.shape
    G = 4
    CH = G * L
    rows_per_w = R * L
    mesh = plsc.VectorSubcoreMesh(core_axis_name="c", subcore_axis_name="s")

    @functools.partial(
        pl.kernel,
        mesh=mesh,
        out_type=jax.ShapeDtypeStruct((NW * rows_per_w, H), jnp.float32),
        scratch_types=[
            pltpu.VMEM((R, L), jnp.int32),
            pltpu.VMEM((CH, H), jnp.float32),
            pltpu.SemaphoreType.DMA,
        ],
    )
    def k(ids_hbm, table_hbm, out_hbm, idx_v, rows_v, sem):
        wid = lax.axis_index("s") * 2 + lax.axis_index("c")
        pltpu.sync_copy(ids_hbm.at[wid], idx_v)

        def body(c, carry):
            cps = [
                pltpu.async_copy(
                    table_hbm.at[idx_v.at[G * c + g]],
                    rows_v.at[pl.ds(g * L, L)], sem)
                for g in range(G)
            ]
            for cp in cps:
                cp.wait()
            pltpu.sync_copy(
                rows_v, out_hbm.at[pl.ds(wid * rows_per_w + c * CH, CH)])
            return carry

        lax.fori_loop(0, R // G, body, 0)

    return k(ids3d, table)


def _fast_sin(x):
    k = jnp.floor(x * 0.15915494309189535 + 0.5)
    r = x - k * 6.2831855
    r2 = r * r
    p = jnp.float32(-0.00014183763)
    p = p * r2 + 0.007904465
    p = p * r2 + -0.16541623
    p = p * r2 + 0.9989872
    return p * r


def _tc_fuse(gathered, age, abspos, tt, params, piece, npieces, out_prev):
    BR = 32
    TB = BR * S

    def _outer(col_v, row_v):
        return lax.dot_general(col_v, row_v, (((1,), (0,)), ((), ())),
                               precision=lax.Precision.DEFAULT)

    def body(g_ref, age_ref, ab_ref, tt_ref, p_ref, out_ref):
        p = p_ref[:]
        wa, ba, wb, bb = p[0:1], p[1:2], p[2:3], p[3:4]
        gamma, beta, seg0, seg1 = p[4:5], p[5:6], p[6:7], p[7:8]
        dseg = seg1 - seg0
        age_t = jnp.transpose(age_ref[:])
        ab_t = jnp.transpose(ab_ref[:])
        tt_t = jnp.transpose(tt_ref[:]).astype(jnp.float32)
        col = lax.broadcasted_iota(jnp.int32, (1, H), 1)
        mones = jnp.full((H, H), 1.0 / H, jnp.float32)
        for j in range(BR):
            sl = pl.ds(j * S, S)
            rows = g_ref[sl, :]
            va = age_t[:, j:j + 1] * wa + ba
            vb = ab_t[:, j:j + 1] * wb + bb
            seg = jnp.where(tt_t[:, j:j + 1] == 0, seg0, seg1)
            t2v = jnp.where(col == 0, va + vb,
                            _fast_sin(va) + _fast_sin(vb))
            emb = rows + seg + t2v
            mean_bc = lax.dot_general(emb, mones, (((1,), (0,)), ((), ())),
                                      precision=lax.Precision.DEFAULT)
            cen = emb - mean_bc
            var_bc = lax.dot_general(cen * cen, mones,
                                     (((1,), (0,)), ((), ())),
                                     precision=lax.Precision.DEFAULT)
            out_ref[sl, :] = cen * lax.rsqrt(var_bc + EPS) * gamma + beta

    nblk, off = piece
    in_specs = [
        pl.BlockSpec((TB, H), lambda i: (i, 0)),
        pl.BlockSpec((BR, S), lambda i: (i + off, 0)),
        pl.BlockSpec((BR, S), lambda i: (i + off, 0)),
        pl.BlockSpec((BR, S), lambda i: (i + off, 0)),
        pl.BlockSpec((8, H), lambda i: (0, 0)),
    ]
    args = [gathered, age, abspos, tt, params]
    aliases = {}
    if out_prev is not None:
        in_specs.append(pl.BlockSpec(memory_space=pl.ANY))
        args.append(out_prev)
        aliases = {5: 0}

    def wrapped(*refs):
        body(*refs[:5], refs[-1])

    return pl.pallas_call(
        wrapped,
        grid=(nblk,),
        in_specs=in_specs,
        out_specs=pl.BlockSpec((TB, H), lambda i: (i + off, 0)),
        out_shape=jax.ShapeDtypeStruct((N, H), jnp.float32),
        input_output_aliases=aliases,
    )(*args)


def kernel(input_ids, token_type_ids, age, abspos, concept_table,
           segment_table, age_w0, age_b0, age_w, age_b,
           abspos_w0, abspos_b0, abspos_w, abspos_b, ln_gamma, ln_beta):
    PIECES = [128, 384, 512]
    L = 100
    params = jnp.stack([
        jnp.concatenate([age_w0, age_w]),
        jnp.concatenate([age_b0, age_b]),
        jnp.concatenate([abspos_w0, abspos_w]),
        jnp.concatenate([abspos_b0, abspos_b]),
        ln_gamma, ln_beta,
        segment_table[0], segment_table[1],
    ])
    ids = input_ids.astype(jnp.int32)
    tt = token_type_ids.astype(jnp.int32)
    gathered = []
    b0 = 0
    for bp in PIECES:
        gathered.append(_sc_gather(
            ids[b0:b0 + bp].reshape(NW, bp * S // NW // L, L),
            concept_table))
        b0 += bp
    out = None
    b0 = 0
    for g, bp in zip(gathered, PIECES):
        out = _tc_fuse(g, age, abspos, tt, params,
                       (bp // 32, b0 // 32), len(PIECES), out)
        b0 += bp
    return out.reshape(B, S, H)


# speedup vs baseline: 11.0684x; 1.0010x over previous
import functools

import jax
import jax.numpy as jnp
from jax import lax
from jax.experimental import pallas as pl
from jax.experimental.pallas import tpu as pltpu
from jax.experimental.pallas import tpu_sc as plsc

B, S, V, T, H = 1024, 200, 100000, 2, 128
N = B * S
EPS = 1e-12

NW = 32
ROWS_PER_W = N // NW
IDXROWS = N // 128
IDXROWS_PER_W = IDXROWS // NW
CHUNK = 256
NCHUNK = ROWS_PER_W // CHUNK


def _sc_gather(ids3d, table):
    _, R, L = ids3d.shape
    G = 4 if R % 4 == 0 else 2
    CH = G * L
    rows_per_w = R * L
    mesh = plsc.VectorSubcoreMesh(core_axis_name="c", subcore_axis_name="s")

    @functools.partial(
        pl.kernel,
        mesh=mesh,
        out_type=jax.ShapeDtypeStruct((NW * rows_per_w, H), jnp.float32),
        scratch_types=[
            pltpu.VMEM((R, L), jnp.int32),
            pltpu.VMEM((CH, H), jnp.float32),
            pltpu.SemaphoreType.DMA,
        ],
    )
    def k(ids_hbm, table_hbm, out_hbm, idx_v, rows_v, sem):
        wid = lax.axis_index("s") * 2 + lax.axis_index("c")
        pltpu.sync_copy(ids_hbm.at[wid], idx_v)

        def body(c, carry):
            cps = [
                pltpu.async_copy(
                    table_hbm.at[idx_v.at[G * c + g]],
                    rows_v.at[pl.ds(g * L, L)], sem)
                for g in range(G)
            ]
            for cp in cps:
                cp.wait()
            pltpu.sync_copy(
                rows_v, out_hbm.at[pl.ds(wid * rows_per_w + c * CH, CH)])
            return carry

        lax.fori_loop(0, R // G, body, 0)

    return k(ids3d, table)


def _fast_sin(x):
    k = jnp.floor(x * 0.15915494309189535 + 0.5)
    r = x - k * 6.2831855
    r2 = r * r
    p = jnp.float32(-0.00014183763)
    p = p * r2 + 0.007904465
    p = p * r2 + -0.16541623
    p = p * r2 + 0.9989872
    return p * r


def _tc_fuse(gathered, age, abspos, tt, params, piece, npieces, out_prev):
    BR = 32
    TB = BR * S

    def _outer(col_v, row_v):
        return lax.dot_general(col_v, row_v, (((1,), (0,)), ((), ())),
                               precision=lax.Precision.DEFAULT)

    def body(g_ref, age_ref, ab_ref, tt_ref, p_ref, out_ref):
        p = p_ref[:]
        wa, ba, wb, bb = p[0:1], p[1:2], p[2:3], p[3:4]
        gamma, beta, seg0, seg1 = p[4:5], p[5:6], p[6:7], p[7:8]
        dseg = seg1 - seg0
        age_t = jnp.transpose(age_ref[:])
        ab_t = jnp.transpose(ab_ref[:])
        tt_t = jnp.transpose(tt_ref[:]).astype(jnp.float32)
        col = lax.broadcasted_iota(jnp.int32, (1, H), 1)
        mones = jnp.full((H, H), 1.0 / H, jnp.float32)
        for j in range(BR):
            sl = pl.ds(j * S, S)
            rows = g_ref[sl, :]
            va = age_t[:, j:j + 1] * wa + ba
            vb = ab_t[:, j:j + 1] * wb + bb
            seg = jnp.where(tt_t[:, j:j + 1] == 0, seg0, seg1)
            t2v = jnp.where(col == 0, va + vb,
                            _fast_sin(va) + _fast_sin(vb))
            emb = rows + seg + t2v
            mean_bc = lax.dot_general(emb, mones, (((1,), (0,)), ((), ())),
                                      precision=lax.Precision.DEFAULT)
            cen = emb - mean_bc
            var_bc = lax.dot_general(cen * cen, mones,
                                     (((1,), (0,)), ((), ())),
                                     precision=lax.Precision.DEFAULT)
            out_ref[sl, :] = cen * lax.rsqrt(var_bc + EPS) * gamma + beta

    nblk, off = piece
    in_specs = [
        pl.BlockSpec((TB, H), lambda i: (i, 0)),
        pl.BlockSpec((BR, S), lambda i: (i + off, 0)),
        pl.BlockSpec((BR, S), lambda i: (i + off, 0)),
        pl.BlockSpec((BR, S), lambda i: (i + off, 0)),
        pl.BlockSpec((8, H), lambda i: (0, 0)),
    ]
    args = [gathered, age, abspos, tt, params]
    aliases = {}
    if out_prev is not None:
        in_specs.append(pl.BlockSpec(memory_space=pl.ANY))
        args.append(out_prev)
        aliases = {5: 0}

    def wrapped(*refs):
        body(*refs[:5], refs[-1])

    return pl.pallas_call(
        wrapped,
        grid=(nblk,),
        in_specs=in_specs,
        out_specs=pl.BlockSpec((TB, H), lambda i: (i + off, 0)),
        out_shape=jax.ShapeDtypeStruct((N, H), jnp.float32),
        input_output_aliases=aliases,
    )(*args)


def kernel(input_ids, token_type_ids, age, abspos, concept_table,
           segment_table, age_w0, age_b0, age_w, age_b,
           abspos_w0, abspos_b0, abspos_w, abspos_b, ln_gamma, ln_beta):
    PIECES = [224, 384, 416]
    L = 100
    params = jnp.stack([
        jnp.concatenate([age_w0, age_w]),
        jnp.concatenate([age_b0, age_b]),
        jnp.concatenate([abspos_w0, abspos_w]),
        jnp.concatenate([abspos_b0, abspos_b]),
        ln_gamma, ln_beta,
        segment_table[0], segment_table[1],
    ])
    ids = input_ids.astype(jnp.int32)
    tt = token_type_ids.astype(jnp.int32)
    gathered = []
    b0 = 0
    for bp in PIECES:
        gathered.append(_sc_gather(
            ids[b0:b0 + bp].reshape(NW, bp * S // NW // L, L),
            concept_table))
        b0 += bp
    out = None
    b0 = 0
    for g, bp in zip(gathered, PIECES):
        out = _tc_fuse(g, age, abspos, tt, params,
                       (bp // 32, b0 // 32), len(PIECES), out)
        b0 += bp
    return out.reshape(B, S, H)


# speedup vs baseline: 11.2107x; 1.0129x over previous
import functools

import jax
import jax.numpy as jnp
from jax import lax
from jax.experimental import pallas as pl
from jax.experimental.pallas import tpu as pltpu
from jax.experimental.pallas import tpu_sc as plsc

B, S, V, T, H = 1024, 200, 100000, 2, 128
N = B * S
EPS = 1e-12

NW = 32
ROWS_PER_W = N // NW
IDXROWS = N // 128
IDXROWS_PER_W = IDXROWS // NW
CHUNK = 256
NCHUNK = ROWS_PER_W // CHUNK


def _sc_gather(ids3d, table):
    _, R, L = ids3d.shape
    G = 4 if R % 4 == 0 else 2
    CH = G * L
    rows_per_w = R * L
    mesh = plsc.VectorSubcoreMesh(core_axis_name="c", subcore_axis_name="s")

    @functools.partial(
        pl.kernel,
        mesh=mesh,
        out_type=jax.ShapeDtypeStruct((NW * rows_per_w, H), jnp.float32),
        scratch_types=[
            pltpu.VMEM((R, L), jnp.int32),
            pltpu.VMEM((CH, H), jnp.float32),
            pltpu.SemaphoreType.DMA,
        ],
    )
    def k(ids_hbm, table_hbm, out_hbm, idx_v, rows_v, sem):
        wid = lax.axis_index("s") * 2 + lax.axis_index("c")
        pltpu.sync_copy(ids_hbm.at[wid], idx_v)

        def body(c, carry):
            cps = [
                pltpu.async_copy(
                    table_hbm.at[idx_v.at[G * c + g]],
                    rows_v.at[pl.ds(g * L, L)], sem)
                for g in range(G)
            ]
            for cp in cps:
                cp.wait()
            pltpu.sync_copy(
                rows_v, out_hbm.at[pl.ds(wid * rows_per_w + c * CH, CH)])
            return carry

        lax.fori_loop(0, R // G, body, 0)

    return k(ids3d, table)


def _fast_sin(x):
    k = jnp.floor(x * 0.15915494309189535 + 0.5)
    r = x - k * 6.2831855
    r2 = r * r
    p = jnp.float32(-0.00014183763)
    p = p * r2 + 0.007904465
    p = p * r2 + -0.16541623
    p = p * r2 + 0.9989872
    return p * r


def _tc_fuse(gathered, age, abspos, tt, params, piece, npieces, out_prev):
    BR = 64
    TB = BR * S

    def _outer(col_v, row_v):
        return lax.dot_general(col_v, row_v, (((1,), (0,)), ((), ())),
                               precision=lax.Precision.DEFAULT)

    def body(g_ref, age_ref, ab_ref, tt_ref, p_ref, out_ref):
        p = p_ref[:]
        wa, ba, wb, bb = p[0:1], p[1:2], p[2:3], p[3:4]
        gamma, beta, seg0, seg1 = p[4:5], p[5:6], p[6:7], p[7:8]
        dseg = seg1 - seg0
        age_t = jnp.transpose(age_ref[:])
        ab_t = jnp.transpose(ab_ref[:])
        tt_t = jnp.transpose(tt_ref[:]).astype(jnp.float32)
        col = lax.broadcasted_iota(jnp.int32, (1, H), 1)
        mones = jnp.full((H, H), 1.0 / H, jnp.float32)
        for j in range(BR):
            sl = pl.ds(j * S, S)
            rows = g_ref[sl, :]
            va = age_t[:, j:j + 1] * wa + ba
            vb = ab_t[:, j:j + 1] * wb + bb
            seg = jnp.where(tt_t[:, j:j + 1] == 0, seg0, seg1)
            t2v = jnp.where(col == 0, va + vb,
                            _fast_sin(va) + _fast_sin(vb))
            emb = rows + seg + t2v
            mean_bc = lax.dot_general(emb, mones, (((1,), (0,)), ((), ())),
                                      precision=lax.Precision.DEFAULT)
            cen = emb - mean_bc
            var_bc = lax.dot_general(cen * cen, mones,
                                     (((1,), (0,)), ((), ())),
                                     precision=lax.Precision.DEFAULT)
            out_ref[sl, :] = cen * lax.rsqrt(var_bc + EPS) * gamma + beta

    nblk, off = piece
    in_specs = [
        pl.BlockSpec((TB, H), lambda i: (i, 0)),
        pl.BlockSpec((BR, S), lambda i: (i + off, 0)),
        pl.BlockSpec((BR, S), lambda i: (i + off, 0)),
        pl.BlockSpec((BR, S), lambda i: (i + off, 0)),
        pl.BlockSpec((8, H), lambda i: (0, 0)),
    ]
    args = [gathered, age, abspos, tt, params]
    aliases = {}
    if out_prev is not None:
        in_specs.append(pl.BlockSpec(memory_space=pl.ANY))
        args.append(out_prev)
        aliases = {5: 0}

    def wrapped(*refs):
        body(*refs[:5], refs[-1])

    return pl.pallas_call(
        wrapped,
        grid=(nblk,),
        in_specs=in_specs,
        out_specs=pl.BlockSpec((TB, H), lambda i: (i + off, 0)),
        out_shape=jax.ShapeDtypeStruct((N, H), jnp.float32),
        input_output_aliases=aliases,
    )(*args)


def kernel(input_ids, token_type_ids, age, abspos, concept_table,
           segment_table, age_w0, age_b0, age_w, age_b,
           abspos_w0, abspos_b0, abspos_w, abspos_b, ln_gamma, ln_beta):
    PIECES = [192, 384, 448]
    L = 100
    params = jnp.stack([
        jnp.concatenate([age_w0, age_w]),
        jnp.concatenate([age_b0, age_b]),
        jnp.concatenate([abspos_w0, abspos_w]),
        jnp.concatenate([abspos_b0, abspos_b]),
        ln_gamma, ln_beta,
        segment_table[0], segment_table[1],
    ])
    ids = input_ids.astype(jnp.int32)
    tt = token_type_ids.astype(jnp.int32)
    gathered = []
    b0 = 0
    for bp in PIECES:
        gathered.append(_sc_gather(
            ids[b0:b0 + bp].reshape(NW, bp * S // NW // L, L),
            concept_table))
        b0 += bp
    out = None
    b0 = 0
    for g, bp in zip(gathered, PIECES):
        out = _tc_fuse(g, age, abspos, tt, params,
                       (bp // 64, b0 // 64), len(PIECES), out)
        b0 += bp
    return out.reshape(B, S, H)
